# Initial kernel scaffold; baseline (speedup 1.0000x reference)
import jax
import jax.numpy as jnp
from jax.experimental import pallas as pl


def kernel(tensor_3d):
    raise NotImplementedError("write your pallas kernel here")


# speedup vs baseline: 86.0997x; 86.0997x over previous
import functools

import jax
import jax.numpy as jnp
from jax import lax
from jax.experimental import pallas as pl
from jax.experimental.pallas import tpu as pltpu
from jax.experimental.pallas import tpu_sc as plsc

FEAT = 1000
NC = 2
NS = 16
NW = NC * NS
L = 16


def _onehot_body(in_hbm, out_hbm, in_vmem, out_vmem, bpw, vals):
    wid = lax.axis_index("s") * NC + lax.axis_index("c")
    pltpu.sync_copy(in_hbm.at[pl.ds(wid * (bpw * vals), bpw * vals)], in_vmem)

    ones = jnp.ones((L,), jnp.float32)
    zeros = jnp.zeros((L,), jnp.float32)
    n_full = vals // L
    tail_off = vals - L

    def batch_body(b, carry):
        obase = b * FEAT
        for j in range(FEAT // L):
            out_vmem[pl.ds(obase + j * L, L)] = zeros
        if FEAT % L:
            out_vmem[pl.ds(obase + FEAT - L, L)] = zeros
        boff = jnp.full((L,), obase, jnp.int32)
        rowbase = b * vals
        for j in range(n_full):
            idx = in_vmem[pl.ds(rowbase + j * L, L)]
            plsc.store_scatter(out_vmem, [idx + boff], ones)
        if vals % L:
            idx = in_vmem[pl.ds(rowbase + tail_off, L)]
            plsc.store_scatter(out_vmem, [idx + boff], ones)
        return carry

    lax.fori_loop(0, bpw, batch_body, 0)
    pltpu.sync_copy(out_vmem, out_hbm.at[pl.ds(wid * (bpw * FEAT), bpw * FEAT)])


def kernel(tensor_3d):
    bsz, d1, d2 = tensor_3d.shape
    vals = d1 * d2
    bpw = bsz // NW
    flat = tensor_3d.astype(jnp.int32).reshape(bsz * vals)
    mesh = plsc.VectorSubcoreMesh(
        core_axis_name="c", subcore_axis_name="s", num_cores=NC, num_subcores=NS
    )
    body = functools.partial(_onehot_body, bpw=bpw, vals=vals)
    f = pl.kernel(
        body,
        out_type=jax.ShapeDtypeStruct((bsz * FEAT,), jnp.float32),
        mesh=mesh,
        scratch_types=[
            pltpu.VMEM((bpw * vals,), jnp.int32),
            pltpu.VMEM((bpw * FEAT,), jnp.float32),
        ],
        compiler_params=pltpu.CompilerParams(needs_layout_passes=False),
    )
    return f(flat).reshape(bsz, FEAT)


# speedup vs baseline: 196.7751x; 2.2854x over previous
import functools

import jax
import jax.numpy as jnp
from jax import lax
from jax.experimental import pallas as pl
from jax.experimental.pallas import tpu as pltpu
from jax.experimental.pallas import tpu_sc as plsc

FEAT = 1000
NC = 2
NS = 16
NW = NC * NS
L = 16


def _onehot_body(in_hbm, out_hbm, in_vmem, out_vmem, bpw, d1, d2):
    wid = lax.axis_index("s") * NC + lax.axis_index("c")
    nfh = FEAT // 8
    nq = 128 // bpw

    pltpu.sync_copy(in_hbm.at[:, :, pl.ds(wid * bpw, bpw)], in_vmem)

    ones = jnp.ones((L,), jnp.float32)
    zeros = jnp.zeros((L,), jnp.float32)

    @plsc.parallel_loop(0, nfh, unroll=2)
    def zero_fh(fh):
        @plsc.parallel_loop(0, 8 * bpw, step=L, unroll=8)
        def zero_i(i):
            out_vmem[fh, i // bpw, pl.ds(i % bpw, L)] = zeros

    lanes0 = lax.iota(jnp.int32, L)

    @plsc.parallel_loop(0, d1)
    def row_body(r):
        @plsc.parallel_loop(0, d2 * (bpw // L), unroll=5)
        def col_body(g):
            c = g // (bpw // L)
            k = g % (bpw // L)
            v = in_vmem[r, c, pl.ds(k * L, L)]
            plsc.store_scatter(
                out_vmem,
                [lax.shift_right_logical(v, 3), lax.bitwise_and(v, 7),
                 lanes0 + k * L],
                ones,
            )

    pltpu.sync_copy(
        out_vmem,
        out_hbm.at[:, wid // nq, :, pl.ds((wid % nq) * bpw, bpw)],
    )


def kernel(tensor_3d):
    bsz, d1, d2 = tensor_3d.shape
    bpw = bsz // NW
    y = jnp.transpose(tensor_3d.astype(jnp.int32), (1, 2, 0))
    mesh = plsc.VectorSubcoreMesh(
        core_axis_name="c", subcore_axis_name="s", num_cores=NC, num_subcores=NS
    )
    body = functools.partial(_onehot_body, bpw=bpw, d1=d1, d2=d2)
    f = pl.kernel(
        body,
        out_type=jax.ShapeDtypeStruct((FEAT // 8, bsz // 128, 8, 128), jnp.float32),
        mesh=mesh,
        scratch_types=[
            pltpu.VMEM((d1, d2, bpw), jnp.int32),
            pltpu.VMEM((FEAT // 8, 8, bpw), jnp.float32),
        ],
        compiler_params=pltpu.CompilerParams(
            needs_layout_passes=False,
            use_tc_tiling_on_sc=False,
            disable_bounds_checks=True,
            disable_semaphore_checks=True,
        ),
    )
    out4 = f(y)
    return out4.transpose(1, 3, 0, 2).reshape(bsz, FEAT)


# speedup vs baseline: 209.1820x; 1.0631x over previous
import functools

import jax
import jax.numpy as jnp
from jax import lax
from jax.experimental import pallas as pl
from jax.experimental.pallas import tpu as pltpu
from jax.experimental.pallas import tpu_sc as plsc

FEAT = 1000
NC = 2
NS = 16
NW = NC * NS
L = 16


def _onehot_body(in_hbm, out_hbm, in_vmem, out_vmem, sem, bpw, d1, d2):
    wid = lax.axis_index("s") * NC + lax.axis_index("c")
    nfh = FEAT // 8
    nq = 128 // bpw

    cp = pltpu.async_copy(in_hbm.at[:, :, pl.ds(wid * bpw, bpw)], in_vmem, sem)

    ones = jnp.ones((L,), jnp.float32)
    zeros = jnp.zeros((L,), jnp.float32)

    @plsc.parallel_loop(0, nfh, unroll=2)
    def zero_fh(fh):
        @plsc.parallel_loop(0, 8 * bpw, step=L, unroll=8)
        def zero_i(i):
            out_vmem[fh, i // bpw, pl.ds(i % bpw, L)] = zeros

    cp.wait()

    lanes0 = lax.iota(jnp.int32, L)

    @plsc.parallel_loop(0, d1)
    def row_body(r):
        @plsc.parallel_loop(0, d2 * (bpw // L), unroll=5)
        def col_body(g):
            c = g // (bpw // L)
            k = g % (bpw // L)
            v = in_vmem[r, c, pl.ds(k * L, L)]
            plsc.store_scatter(
                out_vmem,
                [lax.shift_right_logical(v, 3), lax.bitwise_and(v, 7),
                 lanes0 + k * L],
                ones,
            )

    pltpu.sync_copy(
        out_vmem,
        out_hbm.at[:, wid // nq, :, pl.ds((wid % nq) * bpw, bpw)],
    )


def kernel(tensor_3d):
    bsz, d1, d2 = tensor_3d.shape
    bpw = bsz // NW
    y = jnp.transpose(tensor_3d.astype(jnp.int32), (1, 2, 0))
    mesh = plsc.VectorSubcoreMesh(
        core_axis_name="c", subcore_axis_name="s", num_cores=NC, num_subcores=NS
    )
    body = functools.partial(_onehot_body, bpw=bpw, d1=d1, d2=d2)
    f = pl.kernel(
        body,
        out_type=jax.ShapeDtypeStruct((FEAT // 8, bsz // 128, 8, 128), jnp.float32),
        mesh=mesh,
        scratch_types=[
            pltpu.VMEM((d1, d2, bpw), jnp.int32),
            pltpu.VMEM((FEAT // 8, 8, bpw), jnp.float32),
            pltpu.SemaphoreType.DMA,
        ],
        compiler_params=pltpu.CompilerParams(
            needs_layout_passes=False,
            use_tc_tiling_on_sc=False,
            disable_bounds_checks=True,
            disable_semaphore_checks=True,
        ),
    )
    out4 = f(y)
    return out4.transpose(1, 3, 0, 2).reshape(bsz, FEAT)
